# baseline (speedup 1.0000x reference)
import functools

import jax
import jax.numpy as jnp
from jax import lax
from jax.experimental import pallas as pl
from jax.experimental.pallas import tpu as pltpu
from jax.experimental.pallas import tpu_sc as plsc

_D0, _D1, _D2 = 6, 7, 3
_EMB = 128
_TROWS = 128
_ROWBLK = 25

_NC, _NS = 2, 16
_NW = _NC * _NS
_CHUNK = 80
_NBUF = 5


def _tc_prep_body(ea_ref, w0_ref, w1_ref, w2_ref, t_ref, cidx_ref):
    ea = ea_ref[0]
    m3 = lax.broadcasted_iota(jnp.int32, (1, 3 * _EMB), 1) % 3
    maxv = jnp.where(m3 == 0, _D0 - 1, jnp.where(m3 == 1, _D1 - 1, _D2 - 1))
    wv = jnp.where(m3 == 0, _D1 * _D2, jnp.where(m3 == 1, _D2, 1)).astype(jnp.float32)
    eac = jnp.clip(ea, 0, maxv).astype(jnp.float32) * wv
    rowm = lax.broadcasted_iota(jnp.int32, (3 * _EMB, _EMB), 0) // 3
    colm = lax.broadcasted_iota(jnp.int32, (3 * _EMB, _EMB), 1)
    sel = (rowm == colm).astype(jnp.float32)
    cidx = jnp.dot(eac, sel, preferred_element_type=jnp.float32)
    cidx_ref[...] = cidx.astype(jnp.int32).reshape(1, _ROWBLK, _EMB)

    @pl.when(pl.program_id(0) == 0)
    def _build_table():
        r = lax.broadcasted_iota(jnp.int32, (_TROWS, 1), 0)
        i0 = r // (_D1 * _D2)
        i1 = (r // _D2) % _D1
        i2 = r % _D2
        oh0 = (i0 == lax.broadcasted_iota(jnp.int32, (_TROWS, _D0), 1)).astype(jnp.float32)
        oh1 = (i1 == lax.broadcasted_iota(jnp.int32, (_TROWS, _D1), 1)).astype(jnp.float32)
        oh2 = (i2 == lax.broadcasted_iota(jnp.int32, (_TROWS, _D2), 1)).astype(jnp.float32)
        hi = lax.Precision.HIGHEST
        t_ref[...] = (
            jnp.dot(oh0, w0_ref[...], precision=hi, preferred_element_type=jnp.float32)
            + jnp.dot(oh1, w1_ref[...], precision=hi, preferred_element_type=jnp.float32)
            + jnp.dot(oh2, w2_ref[...], precision=hi, preferred_element_type=jnp.float32)
        )


def _tc_prep(ea3, W0, W1, W2, interpret=False):
    nblk = ea3.shape[0]
    return pl.pallas_call(
        _tc_prep_body,
        grid=(nblk,),
        in_specs=[
            pl.BlockSpec((1, _ROWBLK, 3 * _EMB), lambda i: (i, 0, 0)),
            pl.BlockSpec((_D0, _EMB), lambda i: (0, 0)),
            pl.BlockSpec((_D1, _EMB), lambda i: (0, 0)),
            pl.BlockSpec((_D2, _EMB), lambda i: (0, 0)),
        ],
        out_specs=[
            pl.BlockSpec((_TROWS, _EMB), lambda i: (0, 0)),
            pl.BlockSpec((1, _ROWBLK, _EMB), lambda i: (i, 0, 0)),
        ],
        out_shape=[
            jax.ShapeDtypeStruct((_TROWS, _EMB), jnp.float32),
            jax.ShapeDtypeStruct((nblk, _ROWBLK, _EMB), jnp.int32),
        ],
        interpret=interpret,
    )(ea3, W0, W1, W2)


def _sc_gather(cidx, table):
    n = cidx.shape[0]
    bpw = n // _NW
    nround = bpw // (_CHUNK * _NBUF)
    mesh = plsc.VectorSubcoreMesh(core_axis_name="c", subcore_axis_name="s")

    @functools.partial(
        pl.kernel,
        out_type=jax.ShapeDtypeStruct((n, _EMB), jnp.float32),
        mesh=mesh,
        scratch_types=[
            pltpu.VMEM((bpw,), jnp.int32),
            pltpu.VMEM((_NBUF, _CHUNK, _EMB), jnp.float32),
            pltpu.SemaphoreType.DMA((_NBUF,)),
            pltpu.SemaphoreType.DMA((_NBUF,)),
        ],
    )
    def k(cidx_hbm, t_hbm, out_hbm, idx_v, rows_v, gsem, ssem):
        wid = lax.axis_index("s") * _NC + lax.axis_index("c")
        base = wid * bpw
        pltpu.sync_copy(cidx_hbm.at[pl.ds(base, bpw)], idx_v)

        def ring_round(it, _):
            j0 = it * _NBUF
            handles = []
            for b in range(_NBUF):
                @pl.when(it > 0)
                def _(b=b):
                    pltpu.make_async_copy(
                        rows_v.at[b], out_hbm.at[pl.ds(base, _CHUNK)],
                        ssem.at[b]).wait()
                handles.append(pltpu.async_copy(
                    t_hbm.at[idx_v.at[pl.ds((j0 + b) * _CHUNK, _CHUNK)]],
                    rows_v.at[b], gsem.at[b]))
            for b in range(_NBUF):
                handles[b].wait()
                pltpu.async_copy(
                    rows_v.at[b],
                    out_hbm.at[pl.ds(base + (j0 + b) * _CHUNK, _CHUNK)],
                    ssem.at[b])
            return None

        lax.fori_loop(0, nround, ring_round, None)

        for b in range(_NBUF):
            pltpu.make_async_copy(
                rows_v.at[b], out_hbm.at[pl.ds(base, _CHUNK)], ssem.at[b]).wait()

    return k(cidx, table)


def kernel(edge_attr, W0, W1, W2):
    n = edge_attr.shape[0]
    ea3 = edge_attr.reshape(n // (_ROWBLK * _EMB), _ROWBLK, 3 * _EMB)
    table, cidx = _tc_prep(ea3, W0, W1, W2)
    return _sc_gather(cidx.reshape(-1), table)


# speedup vs baseline: 3.1146x; 3.1146x over previous
import functools

import jax
import jax.numpy as jnp
from jax import lax
from jax.experimental import pallas as pl
from jax.experimental.pallas import tpu as pltpu
from jax.experimental.pallas import tpu_sc as plsc

_D0, _D1, _D2 = 6, 7, 3
_EMB = 128
_TROWS = 128
_ROWBLK = 25

_NC, _NS = 2, 16
_NW = _NC * _NS
_CHUNK = 80
_NBUF = 5


def _tc_prep_body(ea_ref, w0_ref, w1_ref, w2_ref, t_ref, cidx_ref):
    ea = ea_ref[0]
    m3 = lax.broadcasted_iota(jnp.int32, (1, 3 * _EMB), 1) % 3
    maxv = jnp.where(m3 == 0, _D0 - 1, jnp.where(m3 == 1, _D1 - 1, _D2 - 1))
    wv = jnp.where(m3 == 0, _D1 * _D2, jnp.where(m3 == 1, _D2, 1)).astype(jnp.float32)
    eac = jnp.clip(ea, 0, maxv).astype(jnp.float32) * wv
    rowm = lax.broadcasted_iota(jnp.int32, (3 * _EMB, _EMB), 0) // 3
    colm = lax.broadcasted_iota(jnp.int32, (3 * _EMB, _EMB), 1)
    sel = (rowm == colm).astype(jnp.float32)
    cidx = jnp.dot(eac, sel, preferred_element_type=jnp.float32)
    cidx_ref[...] = cidx.astype(jnp.int32).reshape(1, _ROWBLK, _EMB)

    @pl.when(pl.program_id(0) == 0)
    def _build_table():
        r = lax.broadcasted_iota(jnp.int32, (_TROWS, 1), 0)
        i0 = r // (_D1 * _D2)
        i1 = (r // _D2) % _D1
        i2 = r % _D2
        oh0 = (i0 == lax.broadcasted_iota(jnp.int32, (_TROWS, _D0), 1)).astype(jnp.float32)
        oh1 = (i1 == lax.broadcasted_iota(jnp.int32, (_TROWS, _D1), 1)).astype(jnp.float32)
        oh2 = (i2 == lax.broadcasted_iota(jnp.int32, (_TROWS, _D2), 1)).astype(jnp.float32)
        hi = lax.Precision.HIGHEST
        t_ref[...] = (
            jnp.dot(oh0, w0_ref[...], precision=hi, preferred_element_type=jnp.float32)
            + jnp.dot(oh1, w1_ref[...], precision=hi, preferred_element_type=jnp.float32)
            + jnp.dot(oh2, w2_ref[...], precision=hi, preferred_element_type=jnp.float32)
        )


def _tc_prep(ea3, W0, W1, W2, interpret=False):
    nblk = ea3.shape[0]
    return pl.pallas_call(
        _tc_prep_body,
        grid=(nblk,),
        in_specs=[
            pl.BlockSpec((1, _ROWBLK, 3 * _EMB), lambda i: (i, 0, 0)),
            pl.BlockSpec((_D0, _EMB), lambda i: (0, 0)),
            pl.BlockSpec((_D1, _EMB), lambda i: (0, 0)),
            pl.BlockSpec((_D2, _EMB), lambda i: (0, 0)),
        ],
        out_specs=[
            pl.BlockSpec((_TROWS, _EMB), lambda i: (0, 0)),
            pl.BlockSpec((1, _ROWBLK, _EMB), lambda i: (i, 0, 0)),
        ],
        out_shape=[
            jax.ShapeDtypeStruct((_TROWS, _EMB), jnp.float32),
            jax.ShapeDtypeStruct((nblk, _ROWBLK, _EMB), jnp.int32),
        ],
        interpret=interpret,
    )(ea3, W0, W1, W2)


def _sc_gather(cidx, table):
    n = cidx.shape[0]
    bpw = n // _NW
    nround = bpw // (_CHUNK * _NBUF)
    mesh = plsc.VectorSubcoreMesh(core_axis_name="c", subcore_axis_name="s")

    @functools.partial(
        pl.kernel,
        out_type=jax.ShapeDtypeStruct((n, _EMB), jnp.float32),
        mesh=mesh,
        scratch_types=[
            pltpu.VMEM((bpw,), jnp.int32),
            pltpu.VMEM((_NBUF, _CHUNK, _EMB), jnp.float32),
            pltpu.VMEM_SHARED((_TROWS, _EMB), jnp.float32),
            pltpu.SemaphoreType.DMA((_NBUF,)),
            pltpu.SemaphoreType.DMA((_NBUF,)),
        ],
    )
    def k(cidx_hbm, t_hbm, out_hbm, idx_v, rows_v, t_sh, gsem, ssem):
        wid = lax.axis_index("s") * _NC + lax.axis_index("c")
        base = wid * bpw

        @pl.when(lax.axis_index("s") == 0)
        def _stage_table():
            pltpu.sync_copy(t_hbm, t_sh)

        pltpu.sync_copy(cidx_hbm.at[pl.ds(base, bpw)], idx_v)
        plsc.subcore_barrier()

        def ring_round(it, _):
            j0 = it * _NBUF
            handles = []
            for b in range(_NBUF):
                @pl.when(it > 0)
                def _(b=b):
                    pltpu.make_async_copy(
                        rows_v.at[b], out_hbm.at[pl.ds(base, _CHUNK)],
                        ssem.at[b]).wait()
                handles.append(pltpu.async_copy(
                    t_sh.at[idx_v.at[pl.ds((j0 + b) * _CHUNK, _CHUNK)]],
                    rows_v.at[b], gsem.at[b]))
            for b in range(_NBUF):
                handles[b].wait()
                pltpu.async_copy(
                    rows_v.at[b],
                    out_hbm.at[pl.ds(base + (j0 + b) * _CHUNK, _CHUNK)],
                    ssem.at[b])
            return None

        lax.fori_loop(0, nround, ring_round, None)

        for b in range(_NBUF):
            pltpu.make_async_copy(
                rows_v.at[b], out_hbm.at[pl.ds(base, _CHUNK)], ssem.at[b]).wait()

    return k(cidx, table)


def kernel(edge_attr, W0, W1, W2):
    n = edge_attr.shape[0]
    ea3 = edge_attr.reshape(n // (_ROWBLK * _EMB), _ROWBLK, 3 * _EMB)
    table, cidx = _tc_prep(ea3, W0, W1, W2)
    return _sc_gather(cidx.reshape(-1), table)
